# baseline (speedup 1.0000x reference)
import functools

import jax
import jax.numpy as jnp
from jax.experimental import pallas as pl
from jax.experimental.pallas import tpu as pltpu

D = 1024
DFF = 512
E = 64
NG = 8
TKG = 4
TOPK = 8
NSH = 2
RS = 2.5

T = 2048
B = 128
NA = T * TOPK
NPAD_R = NA + E * B
NPAD = NPAD_R + NSH * T
NB_R = NPAD_R // B
NB = NPAD // B


def _gating(x, gate_w, gate_b):
    t = x.shape[0]
    scores = jax.nn.sigmoid(x @ gate_w.T)
    s = scores + gate_b
    sg = s.reshape(t, NG, E // NG)

    lane8 = jnp.arange(E // NG, dtype=jnp.int32)
    m1 = jnp.max(sg, axis=-1, keepdims=True)
    a1 = jnp.min(jnp.where(sg == m1, lane8, E // NG), axis=-1, keepdims=True)
    sg2 = jnp.where(lane8 == a1, -jnp.inf, sg)
    m2 = jnp.max(sg2, axis=-1, keepdims=True)
    group_scores = (m1 + m2)[:, :, 0]

    laneg = jnp.arange(NG, dtype=jnp.int32)
    gs = group_scores
    keep = jnp.zeros((t, NG), dtype=bool)
    for _ in range(TKG):
        m = jnp.max(gs, axis=-1, keepdims=True)
        a = jnp.min(jnp.where(gs == m, laneg, NG), axis=-1, keepdims=True)
        keep = keep | (laneg == a)
        gs = jnp.where(laneg == a, -jnp.inf, gs)

    sm = jnp.where(keep[:, :, None], sg, -jnp.inf).reshape(t, E)

    lanee = jnp.arange(E, dtype=jnp.int32)
    idx_l, wv_l = [], []
    v = sm
    for _ in range(TOPK):
        m = jnp.max(v, axis=-1, keepdims=True)
        a = jnp.min(jnp.where(v == m, lanee, E), axis=-1, keepdims=True)
        onehot = lanee == a
        idx_l.append(a)
        wv_l.append(jnp.sum(jnp.where(onehot, scores, 0.0), axis=-1, keepdims=True))
        v = jnp.where(onehot, -jnp.inf, v)
    sel_oh = (sm != v).astype(jnp.int32)
    idx = jnp.concatenate(idx_l, axis=1)
    w_sel = jnp.concatenate(wv_l, axis=1)
    w_sel = w_sel / w_sel.sum(axis=-1, keepdims=True)
    w_sel = w_sel * RS
    return w_sel, idx, sel_oh


def _expert_block_body(bexp_ref, alive_ref, x_ref, gu_ref, wd_ref,
                       shgu_ref, shwd_ref, wrow_ref, y_ref):
    i = pl.program_id(0)

    @pl.when(alive_ref[i] == 1)
    def _():
        is_sh = bexp_ref[i] >= E
        w1 = jnp.where(is_sh, shgu_ref[0], gu_ref[0]).astype(jnp.bfloat16)
        w2 = jnp.where(is_sh, shwd_ref[0], wd_ref[0]).astype(jnp.bfloat16)
        x = x_ref[...].astype(jnp.bfloat16)
        h = jax.lax.dot_general(x, w1, (((1,), (1,)), ((), ())),
                                preferred_element_type=jnp.float32)
        g = h[:, :DFF]
        u = h[:, DFF:]
        a = (g * jax.nn.sigmoid(g) * u).astype(jnp.bfloat16)
        y = jax.lax.dot_general(a, w2, (((1,), (1,)), ((), ())),
                                preferred_element_type=jnp.float32)
        y_ref[...] = y * wrow_ref[0]


@functools.partial(jax.jit, static_argnums=())
def _grouped_mlp(bexp, alive, x_pad, w_gate_up, w_down, sh_gate_up, sh_down,
                 w_rows):
    grid_spec = pltpu.PrefetchScalarGridSpec(
        num_scalar_prefetch=2,
        grid=(NB,),
        in_specs=[
            pl.BlockSpec((B, D), lambda i, be, al: (i, 0)),
            pl.BlockSpec((1, 2 * DFF, D),
                         lambda i, be, al: (jnp.minimum(be[i], E - 1), 0, 0)),
            pl.BlockSpec((1, D, DFF),
                         lambda i, be, al: (jnp.minimum(be[i], E - 1), 0, 0)),
            pl.BlockSpec((1, 2 * DFF, D),
                         lambda i, be, al: (jnp.clip(be[i] - E, 0, NSH - 1), 0, 0)),
            pl.BlockSpec((1, D, DFF),
                         lambda i, be, al: (jnp.clip(be[i] - E, 0, NSH - 1), 0, 0)),
            pl.BlockSpec((1, B, 1), lambda i, be, al: (i, 0, 0)),
        ],
        out_specs=pl.BlockSpec((B, D), lambda i, be, al: (i, 0)),
    )
    return pl.pallas_call(
        _expert_block_body,
        grid_spec=grid_spec,
        out_shape=jax.ShapeDtypeStruct((NPAD, D), jnp.float32),
    )(bexp, alive, x_pad, w_gate_up, w_down, sh_gate_up, sh_down, w_rows)


def kernel(hidden_states, gate_w, gate_b, w_gate_up, w_down, sh_gate_up, sh_down):
    orig_shape = hidden_states.shape
    x = hidden_states.reshape(-1, D)

    w_sel, idx, sel_oh = _gating(x, gate_w, gate_b)

    e = idx.reshape(-1).astype(jnp.int32)
    csum_in = jnp.cumsum(sel_oh, axis=0)
    counts = csum_in[-1]
    csum_ex = csum_in - sel_oh
    rank = jnp.take_along_axis(csum_ex, idx, axis=1).reshape(-1)
    pc = ((counts + B - 1) // B) * B
    pco = jnp.cumsum(pc)
    po = pco - pc
    posf = (po[e] + rank).astype(jnp.int32)

    tokf = (jnp.arange(NA, dtype=jnp.int32) // TOPK)
    wf = w_sel.reshape(-1)
    tok_pad_r = jnp.zeros((NPAD_R,), jnp.int32).at[posf].set(tokf,
                                                            unique_indices=True)
    w_pad_r = jnp.zeros((NPAD_R,), jnp.float32).at[posf].set(wf,
                                                             unique_indices=True)
    ar_t = jnp.arange(T, dtype=jnp.int32)
    tok_pad = jnp.concatenate([tok_pad_r, ar_t, ar_t])
    w_rows = jnp.concatenate([w_pad_r, jnp.ones((NSH * T,), jnp.float32)])

    blk_start = jnp.arange(NB_R, dtype=jnp.int32) * B
    bexp_r = jnp.searchsorted(pco, blk_start, side='right').astype(jnp.int32)
    alive_r = (blk_start < pco[E - 1]).astype(jnp.int32)
    bexp = jnp.concatenate([
        bexp_r,
        jnp.full((T // B,), E, jnp.int32),
        jnp.full((T // B,), E + 1, jnp.int32),
    ])
    alive = jnp.concatenate([alive_r, jnp.ones((NSH * T // B,), jnp.int32)])

    x_pad = jnp.take(x, tok_pad, axis=0)
    y_pad = _grouped_mlp(bexp, alive, x_pad, w_gate_up, w_down,
                         sh_gate_up, sh_down, w_rows.reshape(NB, B, 1))

    routed = jnp.take(y_pad, posf, axis=0).reshape(T, TOPK, D).sum(axis=1)
    shared = y_pad[NPAD_R:NPAD_R + T] + y_pad[NPAD_R + T:]
    return (routed + shared).reshape(orig_shape)


# speedup vs baseline: 1.3454x; 1.3454x over previous
import functools

import jax
import jax.numpy as jnp
from jax import lax
from jax.experimental import pallas as pl
from jax.experimental.pallas import tpu as pltpu
from jax.experimental.pallas import tpu_sc as plsc

D = 1024
DFF = 512
E = 64
NG = 8
TKG = 4
TOPK = 8
NSH = 2
RS = 2.5

T = 2048
B = 128
NA = T * TOPK
NPAD_R = NA + E * B
NPAD = NPAD_R + NSH * T
NB_R = NPAD_R // B
NB = NPAD // B


def _gating(x, gate_w, gate_b):
    t = x.shape[0]
    scores = jax.nn.sigmoid(x @ gate_w.T)
    s = scores + gate_b
    sg = s.reshape(t, NG, E // NG)

    lane8 = jnp.arange(E // NG, dtype=jnp.int32)
    m1 = jnp.max(sg, axis=-1, keepdims=True)
    a1 = jnp.min(jnp.where(sg == m1, lane8, E // NG), axis=-1, keepdims=True)
    sg2 = jnp.where(lane8 == a1, -jnp.inf, sg)
    m2 = jnp.max(sg2, axis=-1, keepdims=True)
    group_scores = (m1 + m2)[:, :, 0]

    laneg = jnp.arange(NG, dtype=jnp.int32)
    gs = group_scores
    keep = jnp.zeros((t, NG), dtype=bool)
    for _ in range(TKG):
        m = jnp.max(gs, axis=-1, keepdims=True)
        a = jnp.min(jnp.where(gs == m, laneg, NG), axis=-1, keepdims=True)
        keep = keep | (laneg == a)
        gs = jnp.where(laneg == a, -jnp.inf, gs)

    sm = jnp.where(keep[:, :, None], sg, -jnp.inf).reshape(t, E)

    lanee = jnp.arange(E, dtype=jnp.int32)
    idx_l, wv_l = [], []
    v = sm
    for _ in range(TOPK):
        m = jnp.max(v, axis=-1, keepdims=True)
        a = jnp.min(jnp.where(v == m, lanee, E), axis=-1, keepdims=True)
        onehot = lanee == a
        idx_l.append(a)
        wv_l.append(jnp.sum(jnp.where(onehot, scores, 0.0), axis=-1, keepdims=True))
        v = jnp.where(onehot, -jnp.inf, v)
    sel_oh = (sm != v).astype(jnp.int32)
    idx = jnp.concatenate(idx_l, axis=1)
    w_sel = jnp.concatenate(wv_l, axis=1)
    w_sel = w_sel / w_sel.sum(axis=-1, keepdims=True)
    w_sel = w_sel * RS
    return w_sel, idx, sel_oh


def _expert_block_body(bexp_ref, alive_ref, xpad_ref, x2_ref, gu_ref, wd_ref,
                       shgu_ref, shwd_ref, wrow_ref, y_ref):
    i = pl.program_id(0)

    @pl.when(alive_ref[i] == 1)
    def _():
        is_sh = bexp_ref[i] >= E
        w1 = jnp.where(is_sh, shgu_ref[0], gu_ref[0]).astype(jnp.bfloat16)
        w2 = jnp.where(is_sh, shwd_ref[0], wd_ref[0]).astype(jnp.bfloat16)
        x = jnp.where(is_sh, x2_ref[...], xpad_ref[...]).astype(jnp.bfloat16)
        h = jax.lax.dot_general(x, w1, (((1,), (1,)), ((), ())),
                                preferred_element_type=jnp.float32)
        g = h[:, :DFF]
        u = h[:, DFF:]
        a = (g * jax.nn.sigmoid(g) * u).astype(jnp.bfloat16)
        y = jax.lax.dot_general(a, w2, (((1,), (1,)), ((), ())),
                                preferred_element_type=jnp.float32)
        wrow = jnp.where(is_sh, 1.0, wrow_ref[0])
        y_ref[...] = y * wrow


@functools.partial(jax.jit, static_argnums=())
def _grouped_mlp(bexp, alive, x_pad, x, w_gate_up, w_down, sh_gate_up, sh_down,
                 w_rows):
    grid_spec = pltpu.PrefetchScalarGridSpec(
        num_scalar_prefetch=2,
        grid=(NB,),
        in_specs=[
            pl.BlockSpec((B, D), lambda i, be, al: (jnp.minimum(i, NB_R - 1), 0)),
            pl.BlockSpec((B, D), lambda i, be, al: (jnp.maximum(i - NB_R, 0) % (T // B), 0)),
            pl.BlockSpec((1, 2 * DFF, D),
                         lambda i, be, al: (jnp.minimum(be[i], E - 1), 0, 0)),
            pl.BlockSpec((1, D, DFF),
                         lambda i, be, al: (jnp.minimum(be[i], E - 1), 0, 0)),
            pl.BlockSpec((1, 2 * DFF, D),
                         lambda i, be, al: (jnp.clip(be[i] - E, 0, NSH - 1), 0, 0)),
            pl.BlockSpec((1, D, DFF),
                         lambda i, be, al: (jnp.clip(be[i] - E, 0, NSH - 1), 0, 0)),
            pl.BlockSpec((1, B, 1), lambda i, be, al: (jnp.minimum(i, NB_R - 1), 0, 0)),
        ],
        out_specs=pl.BlockSpec((B, D), lambda i, be, al: (i, 0)),
    )
    return pl.pallas_call(
        _expert_block_body,
        grid_spec=grid_spec,
        out_shape=jax.ShapeDtypeStruct((NPAD, D), jnp.float32),
    )(bexp, alive, x_pad, x, w_gate_up, w_down, sh_gate_up, sh_down, w_rows)


_SC_NW = 32
_A_PER_W = NA // _SC_NW
_DCH = 64
_TPW = T // _SC_NW
_TCH = 8
_NFET = TOPK + NSH


def _sc_dispatch_body(x_hbm, tok_hbm, posf_hbm, wf_hbm, xpad_hbm, wpad_hbm,
                      tok_v, dst_v, wv_v, rows_v, sem_g, sem_s, sem_w):
    wid = lax.axis_index("s") * 2 + lax.axis_index("c")

    def chunk(c, carry):
        j0 = wid * _A_PER_W + c * _DCH
        pltpu.sync_copy(tok_hbm.at[pl.ds(j0, _DCH)], tok_v)
        pltpu.sync_copy(posf_hbm.at[pl.ds(j0, _DCH)], dst_v)
        pltpu.sync_copy(wf_hbm.at[pl.ds(j0, _DCH)], wv_v)
        pltpu.async_copy(x_hbm.at[tok_v], rows_v, sem_g).wait()
        pltpu.async_copy(rows_v, xpad_hbm.at[dst_v], sem_s).wait()
        pltpu.async_copy(wv_v, wpad_hbm.at[dst_v], sem_w).wait()
        return carry

    lax.fori_loop(0, _A_PER_W // _DCH, chunk, 0)


_sc_dispatch = pl.kernel(
    _sc_dispatch_body,
    out_type=(jax.ShapeDtypeStruct((NPAD_R, D), jnp.float32),
              jax.ShapeDtypeStruct((NPAD_R,), jnp.float32)),
    mesh=plsc.VectorSubcoreMesh(core_axis_name="c", subcore_axis_name="s", num_cores=2, num_subcores=16),
    scratch_types=[
        pltpu.VMEM((_DCH,), jnp.int32),
        pltpu.VMEM((_DCH,), jnp.int32),
        pltpu.VMEM((_DCH,), jnp.float32),
        pltpu.VMEM((_DCH, D), jnp.float32),
        pltpu.SemaphoreType.DMA,
        pltpu.SemaphoreType.DMA,
        pltpu.SemaphoreType.DMA,
    ],
)


def _sc_combine_body(ypad_hbm, gidx_hbm, out_hbm, idx_v, rows_v, out_v, sem_g):
    wid = lax.axis_index("s") * 2 + lax.axis_index("c")

    def chunk(c, carry):
        rbase = wid * (_TPW * _NFET) + c * (_TCH * _NFET)
        tbase = wid * _TPW + c * _TCH
        pltpu.sync_copy(gidx_hbm.at[pl.ds(rbase, _TCH * _NFET)], idx_v)
        pltpu.async_copy(ypad_hbm.at[idx_v], rows_v, sem_g).wait()

        def tok(t, c2):
            def lanes(cc, c3):
                acc = rows_v[t * _NFET, pl.ds(cc * 16, 16)]
                for r in range(1, _NFET):
                    acc = acc + rows_v[t * _NFET + r, pl.ds(cc * 16, 16)]
                out_v[t, pl.ds(cc * 16, 16)] = acc
                return c3

            lax.fori_loop(0, D // 16, lanes, 0)
            return c2

        lax.fori_loop(0, _TCH, tok, 0)
        pltpu.sync_copy(out_v, out_hbm.at[pl.ds(tbase, _TCH)])
        return carry

    lax.fori_loop(0, _TPW // _TCH, chunk, 0)


_sc_combine = pl.kernel(
    _sc_combine_body,
    out_type=jax.ShapeDtypeStruct((T, D), jnp.float32),
    mesh=plsc.VectorSubcoreMesh(core_axis_name="c", subcore_axis_name="s", num_cores=2, num_subcores=16),
    scratch_types=[
        pltpu.VMEM((_TCH * _NFET,), jnp.int32),
        pltpu.VMEM((_TCH * _NFET, D), jnp.float32),
        pltpu.VMEM((_TCH, D), jnp.float32),
        pltpu.SemaphoreType.DMA,
    ],
)


def kernel(hidden_states, gate_w, gate_b, w_gate_up, w_down, sh_gate_up, sh_down):
    orig_shape = hidden_states.shape
    x = hidden_states.reshape(-1, D)

    w_sel, idx, sel_oh = _gating(x, gate_w, gate_b)

    e = idx.reshape(-1).astype(jnp.int32)
    csum_in = jnp.cumsum(sel_oh, axis=0)
    counts = csum_in[-1]
    csum_ex = csum_in - sel_oh
    rank = jnp.take_along_axis(csum_ex, idx, axis=1).reshape(-1)
    pc = ((counts + B - 1) // B) * B
    pco = jnp.cumsum(pc)
    po = pco - pc
    posf = (po[e] + rank).astype(jnp.int32)

    tokf = (jnp.arange(NA, dtype=jnp.int32) // TOPK)
    wf = w_sel.reshape(-1)

    blk_start = jnp.arange(NB_R, dtype=jnp.int32) * B
    bexp_r = jnp.searchsorted(pco, blk_start, side='right').astype(jnp.int32)
    alive_r = (blk_start < pco[E - 1]).astype(jnp.int32)
    bexp = jnp.concatenate([
        bexp_r,
        jnp.full((T // B,), E, jnp.int32),
        jnp.full((T // B,), E + 1, jnp.int32),
    ])
    alive = jnp.concatenate([alive_r, jnp.ones((NSH * T // B,), jnp.int32)])

    x_pad, w_pad = _sc_dispatch(x, tokf, posf, wf)
    y_pad = _grouped_mlp(bexp, alive, x_pad, x, w_gate_up, w_down,
                         sh_gate_up, sh_down, w_pad.reshape(NB_R, B, 1))

    ar_t = jnp.arange(T, dtype=jnp.int32)
    gidx = jnp.concatenate([
        posf.reshape(T, TOPK),
        (NPAD_R + ar_t)[:, None],
        (NPAD_R + T + ar_t)[:, None],
    ], axis=1).reshape(-1)
    out = _sc_combine(y_pad, gidx)
    return out.reshape(orig_shape)


# speedup vs baseline: 1.6055x; 1.1933x over previous
import functools

import jax
import jax.numpy as jnp
from jax import lax
from jax.experimental import pallas as pl
from jax.experimental.pallas import tpu as pltpu
from jax.experimental.pallas import tpu_sc as plsc

D = 1024
DFF = 512
E = 64
NG = 8
TKG = 4
TOPK = 8
NSH = 2
RS = 2.5

T = 2048
B = 256
NA = T * TOPK
NPAD_R = NA + E * B
NPAD = NPAD_R + NSH * T
NB_R = NPAD_R // B
NB = NPAD // B


def _gating(x, gate_w, gate_b):
    t = x.shape[0]
    scores = jax.nn.sigmoid(x @ gate_w.T)
    s = scores + gate_b
    sg = s.reshape(t, NG, E // NG)

    lane8 = jnp.arange(E // NG, dtype=jnp.int32)
    m1 = jnp.max(sg, axis=-1, keepdims=True)
    a1 = jnp.min(jnp.where(sg == m1, lane8, E // NG), axis=-1, keepdims=True)
    sg2 = jnp.where(lane8 == a1, -jnp.inf, sg)
    m2 = jnp.max(sg2, axis=-1, keepdims=True)
    group_scores = (m1 + m2)[:, :, 0]

    laneg = jnp.arange(NG, dtype=jnp.int32)
    gs = group_scores
    keep = jnp.zeros((t, NG), dtype=bool)
    for _ in range(TKG):
        m = jnp.max(gs, axis=-1, keepdims=True)
        a = jnp.min(jnp.where(gs == m, laneg, NG), axis=-1, keepdims=True)
        keep = keep | (laneg == a)
        gs = jnp.where(laneg == a, -jnp.inf, gs)

    sm = jnp.where(keep[:, :, None], sg, -jnp.inf).reshape(t, E)

    lanee = jnp.arange(E, dtype=jnp.int32)
    idx_l, wv_l = [], []
    v = sm
    for _ in range(TOPK):
        m = jnp.max(v, axis=-1, keepdims=True)
        a = jnp.min(jnp.where(v == m, lanee, E), axis=-1, keepdims=True)
        onehot = lanee == a
        idx_l.append(a)
        wv_l.append(jnp.sum(jnp.where(onehot, scores, 0.0), axis=-1, keepdims=True))
        v = jnp.where(onehot, -jnp.inf, v)
    sel_oh = (sm != v).astype(jnp.int32)
    idx = jnp.concatenate(idx_l, axis=1)
    w_sel = jnp.concatenate(wv_l, axis=1)
    w_sel = w_sel / w_sel.sum(axis=-1, keepdims=True)
    w_sel = w_sel * RS
    return w_sel, idx, sel_oh


def _expert_block_body(bexp_ref, alive_ref, xpad_ref, x2_ref, gu_ref, wd_ref,
                       shgu_ref, shwd_ref, wrow_ref, y_ref):
    i = pl.program_id(0)

    @pl.when(alive_ref[i] == 1)
    def _():
        is_sh = bexp_ref[i] >= E
        w1 = jnp.where(is_sh, shgu_ref[0], gu_ref[0]).astype(jnp.bfloat16)
        w2 = jnp.where(is_sh, shwd_ref[0], wd_ref[0]).astype(jnp.bfloat16)
        x = jnp.where(is_sh, x2_ref[...], xpad_ref[...]).astype(jnp.bfloat16)
        h = jax.lax.dot_general(x, w1, (((1,), (1,)), ((), ())),
                                preferred_element_type=jnp.float32)
        g = h[:, :DFF]
        u = h[:, DFF:]
        a = (g * jax.nn.sigmoid(g) * u).astype(jnp.bfloat16)
        y = jax.lax.dot_general(a, w2, (((1,), (1,)), ((), ())),
                                preferred_element_type=jnp.float32)
        wrow = jnp.where(is_sh, 1.0, wrow_ref[0])
        y_ref[...] = y * wrow


@functools.partial(jax.jit, static_argnums=())
def _grouped_mlp(bexp, alive, x_pad, x, w_gate_up, w_down, sh_gate_up, sh_down,
                 w_rows):
    grid_spec = pltpu.PrefetchScalarGridSpec(
        num_scalar_prefetch=2,
        grid=(NB,),
        in_specs=[
            pl.BlockSpec((B, D), lambda i, be, al: (jnp.minimum(i, NB_R - 1), 0)),
            pl.BlockSpec((B, D), lambda i, be, al: (jnp.maximum(i - NB_R, 0) % (T // B), 0)),
            pl.BlockSpec((1, 2 * DFF, D),
                         lambda i, be, al: (jnp.minimum(be[i], E - 1), 0, 0)),
            pl.BlockSpec((1, D, DFF),
                         lambda i, be, al: (jnp.minimum(be[i], E - 1), 0, 0)),
            pl.BlockSpec((1, 2 * DFF, D),
                         lambda i, be, al: (jnp.clip(be[i] - E, 0, NSH - 1), 0, 0)),
            pl.BlockSpec((1, D, DFF),
                         lambda i, be, al: (jnp.clip(be[i] - E, 0, NSH - 1), 0, 0)),
            pl.BlockSpec((1, B, 1), lambda i, be, al: (jnp.minimum(i, NB_R - 1), 0, 0)),
        ],
        out_specs=pl.BlockSpec((B, D), lambda i, be, al: (i, 0)),
    )
    return pl.pallas_call(
        _expert_block_body,
        grid_spec=grid_spec,
        out_shape=jax.ShapeDtypeStruct((NPAD, D), jnp.float32),
    )(bexp, alive, x_pad, x, w_gate_up, w_down, sh_gate_up, sh_down, w_rows)


_SC_NW = 32
_A_PER_W = NA // _SC_NW
_DCH = 64
_TPW = T // _SC_NW
_TCH = 8
_NFET = TOPK + NSH


def _sc_dispatch_body(x_hbm, tok_hbm, posf_hbm, wf_hbm, xpad_hbm, wpad_hbm,
                      tok_v, dst_v, wv_v, rows_v, sem_g, sem_s, sem_w):
    wid = lax.axis_index("s") * 2 + lax.axis_index("c")

    def chunk(c, carry):
        j0 = wid * _A_PER_W + c * _DCH
        pltpu.sync_copy(tok_hbm.at[pl.ds(j0, _DCH)], tok_v)
        pltpu.sync_copy(posf_hbm.at[pl.ds(j0, _DCH)], dst_v)
        pltpu.sync_copy(wf_hbm.at[pl.ds(j0, _DCH)], wv_v)
        pltpu.async_copy(x_hbm.at[tok_v], rows_v, sem_g).wait()
        pltpu.async_copy(rows_v, xpad_hbm.at[dst_v], sem_s).wait()
        pltpu.async_copy(wv_v, wpad_hbm.at[dst_v], sem_w).wait()
        return carry

    lax.fori_loop(0, _A_PER_W // _DCH, chunk, 0)


_sc_dispatch = pl.kernel(
    _sc_dispatch_body,
    out_type=(jax.ShapeDtypeStruct((NPAD_R, D), jnp.float32),
              jax.ShapeDtypeStruct((NPAD_R,), jnp.float32)),
    mesh=plsc.VectorSubcoreMesh(core_axis_name="c", subcore_axis_name="s", num_cores=2, num_subcores=16),
    scratch_types=[
        pltpu.VMEM((_DCH,), jnp.int32),
        pltpu.VMEM((_DCH,), jnp.int32),
        pltpu.VMEM((_DCH,), jnp.float32),
        pltpu.VMEM((_DCH, D), jnp.float32),
        pltpu.SemaphoreType.DMA,
        pltpu.SemaphoreType.DMA,
        pltpu.SemaphoreType.DMA,
    ],
)


def _sc_combine_body(ypad_hbm, gidx_hbm, out_hbm, idx_v, rows_v, out_v, sem_g):
    wid = lax.axis_index("s") * 2 + lax.axis_index("c")

    def chunk(c, carry):
        rbase = wid * (_TPW * _NFET) + c * (_TCH * _NFET)
        tbase = wid * _TPW + c * _TCH
        pltpu.sync_copy(gidx_hbm.at[pl.ds(rbase, _TCH * _NFET)], idx_v)
        pltpu.async_copy(ypad_hbm.at[idx_v], rows_v, sem_g).wait()

        def tok(t, c2):
            def lanes(cc, c3):
                acc = rows_v[t * _NFET, pl.ds(cc * 16, 16)]
                for r in range(1, _NFET):
                    acc = acc + rows_v[t * _NFET + r, pl.ds(cc * 16, 16)]
                out_v[t, pl.ds(cc * 16, 16)] = acc
                return c3

            lax.fori_loop(0, D // 16, lanes, 0)
            return c2

        lax.fori_loop(0, _TCH, tok, 0)
        pltpu.sync_copy(out_v, out_hbm.at[pl.ds(tbase, _TCH)])
        return carry

    lax.fori_loop(0, _TPW // _TCH, chunk, 0)


_sc_combine = pl.kernel(
    _sc_combine_body,
    out_type=jax.ShapeDtypeStruct((T, D), jnp.float32),
    mesh=plsc.VectorSubcoreMesh(core_axis_name="c", subcore_axis_name="s", num_cores=2, num_subcores=16),
    scratch_types=[
        pltpu.VMEM((_TCH * _NFET,), jnp.int32),
        pltpu.VMEM((_TCH * _NFET, D), jnp.float32),
        pltpu.VMEM((_TCH, D), jnp.float32),
        pltpu.SemaphoreType.DMA,
    ],
)


def kernel(hidden_states, gate_w, gate_b, w_gate_up, w_down, sh_gate_up, sh_down):
    orig_shape = hidden_states.shape
    x = hidden_states.reshape(-1, D)

    w_sel, idx, sel_oh = _gating(x, gate_w, gate_b)

    e = idx.reshape(-1).astype(jnp.int32)
    csum_in = jnp.cumsum(sel_oh, axis=0)
    counts = csum_in[-1]
    csum_ex = csum_in - sel_oh
    rank = jnp.take_along_axis(csum_ex, idx, axis=1).reshape(-1)
    pc = ((counts + B - 1) // B) * B
    pco = jnp.cumsum(pc)
    po = pco - pc
    posf = (po[e] + rank).astype(jnp.int32)

    tokf = (jnp.arange(NA, dtype=jnp.int32) // TOPK)
    wf = w_sel.reshape(-1)

    blk_start = jnp.arange(NB_R, dtype=jnp.int32) * B
    bexp_r = jnp.searchsorted(pco, blk_start, side='right').astype(jnp.int32)
    alive_r = (blk_start < pco[E - 1]).astype(jnp.int32)
    bexp = jnp.concatenate([
        bexp_r,
        jnp.full((T // B,), E, jnp.int32),
        jnp.full((T // B,), E + 1, jnp.int32),
    ])
    alive = jnp.concatenate([alive_r, jnp.ones((NSH * T // B,), jnp.int32)])

    x_pad, w_pad = _sc_dispatch(x, tokf, posf, wf)
    y_pad = _grouped_mlp(bexp, alive, x_pad, x, w_gate_up, w_down,
                         sh_gate_up, sh_down, w_pad.reshape(NB_R, B, 1))

    ar_t = jnp.arange(T, dtype=jnp.int32)
    gidx = jnp.concatenate([
        posf.reshape(T, TOPK),
        (NPAD_R + ar_t)[:, None],
        (NPAD_R + T + ar_t)[:, None],
    ], axis=1).reshape(-1)
    out = _sc_combine(y_pad, gidx)
    return out.reshape(orig_shape)


# speedup vs baseline: 1.6766x; 1.0443x over previous
import functools

import jax
import jax.numpy as jnp
from jax import lax
from jax.experimental import pallas as pl
from jax.experimental.pallas import tpu as pltpu
from jax.experimental.pallas import tpu_sc as plsc

D = 1024
DFF = 512
E = 64
NG = 8
TKG = 4
TOPK = 8
NSH = 2
RS = 2.5

T = 2048
B = 256
NA = T * TOPK
NPAD_R = NA + E * B
NPAD = NPAD_R + NSH * T
NB_R = NPAD_R // B
NB = NPAD // B


def _gating(x, gate_w, gate_b):
    t = x.shape[0]
    scores = jax.nn.sigmoid(x @ gate_w.T)
    s = scores + gate_b
    sg = s.reshape(t, NG, E // NG)

    lane8 = jnp.arange(E // NG, dtype=jnp.int32)
    m1 = jnp.max(sg, axis=-1, keepdims=True)
    a1 = jnp.min(jnp.where(sg == m1, lane8, E // NG), axis=-1, keepdims=True)
    sg2 = jnp.where(lane8 == a1, -jnp.inf, sg)
    m2 = jnp.max(sg2, axis=-1, keepdims=True)
    group_scores = (m1 + m2)[:, :, 0]

    laneg = jnp.arange(NG, dtype=jnp.int32)
    gs = group_scores
    keep = jnp.zeros((t, NG), dtype=bool)
    for _ in range(TKG):
        m = jnp.max(gs, axis=-1, keepdims=True)
        a = jnp.min(jnp.where(gs == m, laneg, NG), axis=-1, keepdims=True)
        keep = keep | (laneg == a)
        gs = jnp.where(laneg == a, -jnp.inf, gs)

    sm = jnp.where(keep[:, :, None], sg, -jnp.inf).reshape(t, E)

    lanee = jnp.arange(E, dtype=jnp.int32)
    idx_l, wv_l = [], []
    v = sm
    for _ in range(TOPK):
        m = jnp.max(v, axis=-1, keepdims=True)
        a = jnp.min(jnp.where(v == m, lanee, E), axis=-1, keepdims=True)
        onehot = lanee == a
        idx_l.append(a)
        wv_l.append(jnp.sum(jnp.where(onehot, scores, 0.0), axis=-1, keepdims=True))
        v = jnp.where(onehot, -jnp.inf, v)
    sel_oh = (sm != v).astype(jnp.int32)
    idx = jnp.concatenate(idx_l, axis=1)
    w_sel = jnp.concatenate(wv_l, axis=1)
    w_sel = w_sel / w_sel.sum(axis=-1, keepdims=True)
    w_sel = w_sel * RS
    return w_sel, idx, sel_oh


def _expert_block_body(bexp_ref, alive_ref, xpad_ref, x2_ref, gu_ref, wd_ref,
                       shgu_ref, shwd_ref, wrow_ref, y_ref):
    i = pl.program_id(0)

    @pl.when(alive_ref[i] == 1)
    def _():
        is_sh = bexp_ref[i] >= E
        w1 = jnp.where(is_sh, shgu_ref[0], gu_ref[0]).astype(jnp.bfloat16)
        w2 = jnp.where(is_sh, shwd_ref[0], wd_ref[0]).astype(jnp.bfloat16)
        x = jnp.where(is_sh, x2_ref[...], xpad_ref[...]).astype(jnp.bfloat16)
        h = jax.lax.dot_general(x, w1, (((1,), (1,)), ((), ())),
                                preferred_element_type=jnp.float32)
        g = h[:, :DFF]
        u = h[:, DFF:]
        a = (g * jax.nn.sigmoid(g) * u).astype(jnp.bfloat16)
        y = jax.lax.dot_general(a, w2, (((1,), (1,)), ((), ())),
                                preferred_element_type=jnp.float32)
        wrow = jnp.where(is_sh, 1.0, wrow_ref[0])
        y_ref[...] = y * wrow


@functools.partial(jax.jit, static_argnums=())
def _grouped_mlp(bexp, alive, x_pad, x, w_gate_up, w_down, sh_gate_up, sh_down,
                 w_rows):
    grid_spec = pltpu.PrefetchScalarGridSpec(
        num_scalar_prefetch=2,
        grid=(NB,),
        in_specs=[
            pl.BlockSpec((B, D), lambda i, be, al: (jnp.minimum(i, NB_R - 1), 0)),
            pl.BlockSpec((B, D), lambda i, be, al: (jnp.maximum(i - NB_R, 0) % (T // B), 0)),
            pl.BlockSpec((1, 2 * DFF, D),
                         lambda i, be, al: (jnp.minimum(be[i], E - 1), 0, 0)),
            pl.BlockSpec((1, D, DFF),
                         lambda i, be, al: (jnp.minimum(be[i], E - 1), 0, 0)),
            pl.BlockSpec((1, 2 * DFF, D),
                         lambda i, be, al: (jnp.clip(be[i] - E, 0, NSH - 1), 0, 0)),
            pl.BlockSpec((1, D, DFF),
                         lambda i, be, al: (jnp.clip(be[i] - E, 0, NSH - 1), 0, 0)),
            pl.BlockSpec((1, B, 1), lambda i, be, al: (jnp.minimum(i, NB_R - 1), 0, 0)),
        ],
        out_specs=pl.BlockSpec((B, D), lambda i, be, al: (i, 0)),
    )
    return pl.pallas_call(
        _expert_block_body,
        grid_spec=grid_spec,
        out_shape=jax.ShapeDtypeStruct((NPAD, D), jnp.float32),
    )(bexp, alive, x_pad, x, w_gate_up, w_down, sh_gate_up, sh_down, w_rows)


_SC_NW = 32
_A_PER_W = NA // _SC_NW
_DCH = 32
_TPW = T // _SC_NW
_TCH = 4
_NFET = TOPK + NSH


def _sc_dispatch_body(x_hbm, tok_hbm, posf_hbm, wf_hbm, xpad_hbm, wpad_hbm,
                      tok_v0, dst_v0, wv_v0, rows_v0,
                      tok_v1, dst_v1, wv_v1, rows_v1, sem_g, sem_s, sem_w):
    wid = lax.axis_index("s") * 2 + lax.axis_index("c")
    bufs = [(tok_v0, dst_v0, wv_v0, rows_v0), (tok_v1, dst_v1, wv_v1, rows_v1)]
    nch = _A_PER_W // _DCH
    g_d = [None] * nch
    s_d = [None] * nch
    w_d = [None] * nch
    for c in range(nch):
        tok_v, dst_v, wv_v, rows_v = bufs[c % 2]
        if c >= 2:
            s_d[c - 2].wait()
            w_d[c - 2].wait()
        j0 = wid * _A_PER_W + c * _DCH
        pltpu.sync_copy(tok_hbm.at[pl.ds(j0, _DCH)], tok_v)
        pltpu.sync_copy(posf_hbm.at[pl.ds(j0, _DCH)], dst_v)
        pltpu.sync_copy(wf_hbm.at[pl.ds(j0, _DCH)], wv_v)
        g_d[c] = pltpu.async_copy(x_hbm.at[tok_v], rows_v, sem_g)
        if c >= 1:
            ptok, pdst, pwv, prows = bufs[(c - 1) % 2]
            g_d[c - 1].wait()
            s_d[c - 1] = pltpu.async_copy(prows, xpad_hbm.at[pdst], sem_s)
            w_d[c - 1] = pltpu.async_copy(pwv, wpad_hbm.at[pdst], sem_w)
    ltok, ldst, lwv, lrows = bufs[(nch - 1) % 2]
    g_d[nch - 1].wait()
    s_d[nch - 1] = pltpu.async_copy(lrows, xpad_hbm.at[ldst], sem_s)
    w_d[nch - 1] = pltpu.async_copy(lwv, wpad_hbm.at[ldst], sem_w)
    for c in (nch - 2, nch - 1):
        s_d[c].wait()
        w_d[c].wait()


_sc_dispatch = pl.kernel(
    _sc_dispatch_body,
    out_type=(jax.ShapeDtypeStruct((NPAD_R, D), jnp.float32),
              jax.ShapeDtypeStruct((NPAD_R,), jnp.float32)),
    mesh=plsc.VectorSubcoreMesh(core_axis_name="c", subcore_axis_name="s", num_cores=2, num_subcores=16),
    scratch_types=[
        pltpu.VMEM((_DCH,), jnp.int32),
        pltpu.VMEM((_DCH,), jnp.int32),
        pltpu.VMEM((_DCH,), jnp.float32),
        pltpu.VMEM((_DCH, D), jnp.float32),
        pltpu.VMEM((_DCH,), jnp.int32),
        pltpu.VMEM((_DCH,), jnp.int32),
        pltpu.VMEM((_DCH,), jnp.float32),
        pltpu.VMEM((_DCH, D), jnp.float32),
        pltpu.SemaphoreType.DMA,
        pltpu.SemaphoreType.DMA,
        pltpu.SemaphoreType.DMA,
    ],
)


def _sc_combine_body(ypad_hbm, gidx_hbm, out_hbm, idx_v0, rows_v0,
                     idx_v1, rows_v1, out_v, sem_g):
    wid = lax.axis_index("s") * 2 + lax.axis_index("c")
    bufs = [(idx_v0, rows_v0), (idx_v1, rows_v1)]
    nch = _TPW // _TCH
    g_d = [None] * nch

    def accum(rows_v, tbase):
        def tok(t, c2):
            def lanes(cc, c3):
                acc = rows_v[t * _NFET, pl.ds(cc * 16, 16)]
                for r in range(1, _NFET):
                    acc = acc + rows_v[t * _NFET + r, pl.ds(cc * 16, 16)]
                out_v[t, pl.ds(cc * 16, 16)] = acc
                return c3

            lax.fori_loop(0, D // 16, lanes, 0)
            return c2

        lax.fori_loop(0, _TCH, tok, 0)
        pltpu.sync_copy(out_v, out_hbm.at[pl.ds(tbase, _TCH)])

    for c in range(nch):
        idx_v, rows_v = bufs[c % 2]
        rbase = wid * (_TPW * _NFET) + c * (_TCH * _NFET)
        pltpu.sync_copy(gidx_hbm.at[pl.ds(rbase, _TCH * _NFET)], idx_v)
        g_d[c] = pltpu.async_copy(ypad_hbm.at[idx_v], rows_v, sem_g)
        if c >= 1:
            g_d[c - 1].wait()
            accum(bufs[(c - 1) % 2][1], wid * _TPW + (c - 1) * _TCH)
    g_d[nch - 1].wait()
    accum(bufs[(nch - 1) % 2][1], wid * _TPW + (nch - 1) * _TCH)


_sc_combine = pl.kernel(
    _sc_combine_body,
    out_type=jax.ShapeDtypeStruct((T, D), jnp.float32),
    mesh=plsc.VectorSubcoreMesh(core_axis_name="c", subcore_axis_name="s", num_cores=2, num_subcores=16),
    scratch_types=[
        pltpu.VMEM((_TCH * _NFET,), jnp.int32),
        pltpu.VMEM((_TCH * _NFET, D), jnp.float32),
        pltpu.VMEM((_TCH * _NFET,), jnp.int32),
        pltpu.VMEM((_TCH * _NFET, D), jnp.float32),
        pltpu.VMEM((_TCH, D), jnp.float32),
        pltpu.SemaphoreType.DMA,
    ],
)


def kernel(hidden_states, gate_w, gate_b, w_gate_up, w_down, sh_gate_up, sh_down):
    orig_shape = hidden_states.shape
    x = hidden_states.reshape(-1, D)

    w_sel, idx, sel_oh = _gating(x, gate_w, gate_b)

    e = idx.reshape(-1).astype(jnp.int32)
    csum_in = jnp.cumsum(sel_oh, axis=0)
    counts = csum_in[-1]
    csum_ex = csum_in - sel_oh
    rank = jnp.take_along_axis(csum_ex, idx, axis=1).reshape(-1)
    pc = ((counts + B - 1) // B) * B
    pco = jnp.cumsum(pc)
    po = pco - pc
    posf = (po[e] + rank).astype(jnp.int32)

    tokf = (jnp.arange(NA, dtype=jnp.int32) // TOPK)
    wf = w_sel.reshape(-1)

    blk_start = jnp.arange(NB_R, dtype=jnp.int32) * B
    bexp_r = jnp.searchsorted(pco, blk_start, side='right').astype(jnp.int32)
    alive_r = (blk_start < pco[E - 1]).astype(jnp.int32)
    bexp = jnp.concatenate([
        bexp_r,
        jnp.full((T // B,), E, jnp.int32),
        jnp.full((T // B,), E + 1, jnp.int32),
    ])
    alive = jnp.concatenate([alive_r, jnp.ones((NSH * T // B,), jnp.int32)])

    x_pad, w_pad = _sc_dispatch(x, tokf, posf, wf)
    y_pad = _grouped_mlp(bexp, alive, x_pad, x, w_gate_up, w_down,
                         sh_gate_up, sh_down, w_pad.reshape(NB_R, B, 1))

    ar_t = jnp.arange(T, dtype=jnp.int32)
    gidx = jnp.concatenate([
        posf.reshape(T, TOPK),
        (NPAD_R + ar_t)[:, None],
        (NPAD_R + T + ar_t)[:, None],
    ], axis=1).reshape(-1)
    out = _sc_combine(y_pad, gidx)
    return out.reshape(orig_shape)
